# baseline (speedup 1.0000x reference)
import jax
import jax.numpy as jnp
from jax.experimental import pallas as pl
from jax.experimental.pallas import tpu as pltpu

_SLAB = 1024
_NBUF = 6


def _vq_logits_kernel(keys_ref, emb_ref, out_ref, stage_ref, sems):
    i = pl.program_id(0)
    nslab = pl.num_programs(0)
    j = jax.lax.rem(i, _NBUF)

    def slab_copy(step, buf):
        return pltpu.make_async_copy(
            stage_ref.at[buf],
            out_ref.at[pl.ds(step * _SLAB, _SLAB), :],
            sems.at[buf])

    @pl.when(i >= _NBUF)
    def _():
        slab_copy(i - _NBUF, j).wait()

    keys = keys_ref[...]
    emb = emb_ref[...]
    cross = jax.lax.dot_general(
        keys.astype(jnp.bfloat16), emb.astype(jnp.bfloat16),
        (((1,), (1,)), ((), ())),
        preferred_element_type=jnp.float32)
    k_sq = jnp.sum(keys * keys, axis=1, keepdims=True)
    e_sq = jnp.sum(emb * emb, axis=1)[None, :]
    stage_ref[j] = (2.0 * cross - k_sq) - e_sq
    slab_copy(i, j).start()

    @pl.when(i == nslab - 1)
    def _():
        def drain(step, carry):
            slab_copy(step, jax.lax.rem(step, _NBUF)).wait()
            return carry
        jax.lax.fori_loop(jnp.maximum(0, nslab - _NBUF), nslab, drain, 0)


def kernel(keys, embeddings):
    B, C = keys.shape
    K = embeddings.shape[0]
    return pl.pallas_call(
        _vq_logits_kernel,
        grid=(B // _SLAB,),
        in_specs=[
            pl.BlockSpec((_SLAB, C), lambda i: (i, 0)),
            pl.BlockSpec((K, C), lambda i: (0, 0)),
        ],
        out_specs=pl.BlockSpec(memory_space=pl.ANY),
        out_shape=jax.ShapeDtypeStruct((B, K), jnp.float32),
        scratch_shapes=[
            pltpu.VMEM((_NBUF, _SLAB, K), jnp.float32),
            pltpu.SemaphoreType.DMA((_NBUF,)),
        ],
    )(keys, embeddings)


# speedup vs baseline: 1.0326x; 1.0326x over previous
import jax
import jax.numpy as jnp
from jax.experimental import pallas as pl
from jax.experimental.pallas import tpu as pltpu

_SLAB = 1024
_NBUF = 6


def _vq_logits_kernel(keys_ref, emb_ref, out_ref, stage_ref, sems):
    nslab = keys_ref.shape[0] // _SLAB
    emb = emb_ref[...]
    emb_bf = emb.astype(jnp.bfloat16)
    e_sq = jnp.sum(emb * emb, axis=1)[None, :]

    def slab_copy(step, buf):
        return pltpu.make_async_copy(
            stage_ref.at[buf],
            out_ref.at[pl.ds(step * _SLAB, _SLAB), :],
            sems.at[buf])

    for i in range(nslab):
        j = i % _NBUF
        if i >= _NBUF:
            slab_copy(i - _NBUF, j).wait()
        keys = keys_ref[pl.ds(i * _SLAB, _SLAB), :]
        cross = jax.lax.dot_general(
            keys.astype(jnp.bfloat16), emb_bf,
            (((1,), (1,)), ((), ())),
            preferred_element_type=jnp.float32)
        k_sq = jnp.sum(keys * keys, axis=1, keepdims=True)
        stage_ref[j] = (2.0 * cross - k_sq) - e_sq
        slab_copy(i, j).start()

    for i in range(max(0, nslab - _NBUF), nslab):
        slab_copy(i, i % _NBUF).wait()


def kernel(keys, embeddings):
    B, C = keys.shape
    K = embeddings.shape[0]
    return pl.pallas_call(
        _vq_logits_kernel,
        in_specs=[
            pl.BlockSpec(memory_space=pltpu.MemorySpace.VMEM),
            pl.BlockSpec(memory_space=pltpu.MemorySpace.VMEM),
        ],
        out_specs=pl.BlockSpec(memory_space=pl.ANY),
        out_shape=jax.ShapeDtypeStruct((B, K), jnp.float32),
        scratch_shapes=[
            pltpu.VMEM((_NBUF, _SLAB, K), jnp.float32),
            pltpu.SemaphoreType.DMA((_NBUF,)),
        ],
    )(keys, embeddings)


# speedup vs baseline: 1.0496x; 1.0165x over previous
import jax
import jax.numpy as jnp
from jax.experimental import pallas as pl
from jax.experimental.pallas import tpu as pltpu

_BM = 3072


def _vq_logits_kernel(keys_ref, emb_ref, out_ref):
    keys = keys_ref[...]
    emb = emb_ref[...]
    bm = keys.shape[0]
    kk = emb.shape[0]
    f32 = jnp.float32
    bf16 = jnp.bfloat16

    k_sq = jnp.sum(keys * keys, axis=1, keepdims=True)
    k_hi = k_sq.astype(bf16)
    k_lo = (k_sq - k_hi.astype(f32)).astype(bf16)
    ones_b = jnp.ones((bm, 1), bf16)
    a = jnp.concatenate(
        [(keys + keys).astype(bf16), k_hi, k_lo, ones_b, ones_b], axis=1)

    e_sq = jnp.sum(emb * emb, axis=1, keepdims=True)
    e_hi = e_sq.astype(bf16)
    e_lo = (e_sq - e_hi.astype(f32)).astype(bf16)
    neg_ones_k = jnp.full((kk, 1), -1, bf16)
    b = jnp.concatenate(
        [emb.astype(bf16), neg_ones_k, neg_ones_k, -e_hi, -e_lo], axis=1)

    out_ref[...] = jax.lax.dot_general(
        a, b, (((1,), (1,)), ((), ())),
        preferred_element_type=f32)


def kernel(keys, embeddings):
    B, C = keys.shape
    K = embeddings.shape[0]
    return pl.pallas_call(
        _vq_logits_kernel,
        grid=(B // _BM,),
        in_specs=[
            pl.BlockSpec((_BM, C), lambda i: (i, 0)),
            pl.BlockSpec((K, C), lambda i: (0, 0)),
        ],
        out_specs=pl.BlockSpec((_BM, K), lambda i: (i, 0)),
        out_shape=jax.ShapeDtypeStruct((B, K), jnp.float32),
        compiler_params=pltpu.CompilerParams(
            dimension_semantics=("parallel",)),
    )(keys, embeddings)
